# speedup vs baseline: 2.8664x; 2.8664x over previous
import jax
import jax.numpy as jnp
from jax.experimental import pallas as pl
from jax.experimental.pallas import tpu as pltpu
from jax.experimental.pallas import tpu_sc as plsc

_W = 128


def kernel(x, I, pe):
    batch, hist = x.shape
    dim = I.shape[1]
    n = batch * hist
    idx = x.reshape(1, n)

    mesh = plsc.VectorSubcoreMesh(core_axis_name="core",
                                  subcore_axis_name="subcore")

    @pl.kernel(out_type=jax.ShapeDtypeStruct((n, dim), I.dtype), mesh=mesh)
    def gather_kernel(table_hbm, i_hbm, o_hbm):
        def body(i_vmem, o_vmem):
            pltpu.sync_copy(table_hbm.at[i_vmem.at[0]], o_vmem)

        pltpu.emit_pipeline(
            body,
            grid=(n // _W,),
            in_specs=[pl.BlockSpec((1, _W), index_map=lambda i: (0, i))],
            out_specs=[pl.BlockSpec((_W, dim), index_map=lambda i: (i, 0))],
            core_axis_name=("core", "subcore"),
            dimension_semantics=(pltpu.PARALLEL,),
        )(i_hbm, o_hbm)

    out = gather_kernel(I, idx)
    return out.reshape(batch, hist, dim)


# speedup vs baseline: 2.8684x; 1.0007x over previous
import jax
import jax.numpy as jnp
from jax.experimental import pallas as pl
from jax.experimental.pallas import tpu as pltpu
from jax.experimental.pallas import tpu_sc as plsc

_G = 128
_K = 2
_W = _G * _K


def kernel(x, I, pe):
    batch, hist = x.shape
    dim = I.shape[1]
    n = batch * hist
    idx = x.reshape(n // _G, _G)

    mesh = plsc.VectorSubcoreMesh(core_axis_name="core",
                                  subcore_axis_name="subcore")

    @pl.kernel(out_type=jax.ShapeDtypeStruct((n, dim), I.dtype), mesh=mesh)
    def gather_kernel(table_hbm, i_hbm, o_hbm):
        def body(i_vmem, o_vmem):
            for j in range(_K):
                pltpu.sync_copy(table_hbm.at[i_vmem.at[j]],
                                o_vmem.at[pl.ds(j * _G, _G)])

        pltpu.emit_pipeline(
            body,
            grid=(n // _W,),
            in_specs=[pl.BlockSpec((_K, _G), index_map=lambda i: (i, 0))],
            out_specs=[pl.BlockSpec((_W, dim), index_map=lambda i: (i, 0))],
            core_axis_name=("core", "subcore"),
            dimension_semantics=(pltpu.PARALLEL,),
        )(i_hbm, o_hbm)

    out = gather_kernel(I, idx)
    return out.reshape(batch, hist, dim)


# speedup vs baseline: 14.9370x; 5.2074x over previous
import jax
import jax.numpy as jnp
from jax.experimental import pallas as pl
from jax.experimental.pallas import tpu as pltpu
from jax.experimental.pallas import tpu_sc as plsc

_G = 128
_K = 2
_W = _G * _K


def kernel(x, I, pe):
    batch, hist = x.shape
    dim = I.shape[1]
    n = batch * hist
    idx = x.reshape(n // _G, _G)

    mesh = plsc.VectorSubcoreMesh(core_axis_name="core",
                                  subcore_axis_name="subcore")

    @pl.kernel(out_type=jax.ShapeDtypeStruct((n, dim), I.dtype), mesh=mesh,
               scratch_types=[pltpu.VMEM_SHARED((128, 128), jnp.float32)])
    def gather_kernel(table_hbm, i_hbm, o_hbm, table_sh):
        sid = jax.lax.axis_index("subcore")

        @pl.when(sid == 0)
        def _():
            pltpu.sync_copy(table_hbm, table_sh)

        plsc.subcore_barrier()

        def body(i_vmem, o_vmem):
            for j in range(_K):
                pltpu.sync_copy(table_sh.at[i_vmem.at[j]],
                                o_vmem.at[pl.ds(j * _G, _G)])

        pltpu.emit_pipeline(
            body,
            grid=(n // _W,),
            in_specs=[pl.BlockSpec((_K, _G), index_map=lambda i: (i, 0))],
            out_specs=[pl.BlockSpec((_W, dim), index_map=lambda i: (i, 0))],
            core_axis_name=("core", "subcore"),
            dimension_semantics=(pltpu.PARALLEL,),
        )(i_hbm, o_hbm)

    out = gather_kernel(I, idx)
    return out.reshape(batch, hist, dim)
